# baseline (speedup 1.0000x reference)
import functools

import jax
import jax.numpy as jnp
import numpy as np
from jax.experimental import pallas as pl
from jax.experimental.pallas import tpu as pltpu

B = 64
H = 64
D = 128
S = 4096
TOPK = 2048
S_BLK = 2048


def _hadamard_np(d):
    h = np.array([[1.0]], dtype=np.float32)
    while h.shape[0] < d:
        h = np.block([[h, h], [h, -h]])
    return (h * (d ** -0.5)).astype(np.float32)

_HAD_CONST = _hadamard_np(D)


def _logits_body(kv_lens_ref, q_ref, k_ref, w_ref, had_ref, out_ref):
    sblk = pl.program_id(1)
    b = pl.program_id(0)
    had = had_ref[...]
    qm = jax.lax.dot_general(
        q_ref[0], had, (((1,), (0,)), ((), ())),
        preferred_element_type=jnp.float32)
    km = jax.lax.dot_general(
        k_ref[0], had, (((1,), (0,)), ((), ())),
        preferred_element_type=jnp.float32)
    scores = jax.lax.dot_general(
        qm, km, (((1,), (1,)), ((), ())), preferred_element_type=jnp.float32)
    scores = jnp.maximum(scores, 0.0) * (D ** -0.5)
    w = w_ref[0]
    logits = jax.lax.dot_general(
        w, scores, (((1,), (0,)), ((), ())), preferred_element_type=jnp.float32)
    pos = sblk * S_BLK + jax.lax.broadcasted_iota(jnp.int32, (1, S_BLK), 1)
    kv_len = kv_lens_ref[b]
    masked_key = -1.0 - pos.astype(jnp.float32)
    keys = jnp.where(pos < kv_len, logits, masked_key)
    out_ref[...] = keys.reshape(1, 1, 1, S_BLK)


def _compute_keys(q, k_cache, weights, kv_lens):
    grid_spec = pltpu.PrefetchScalarGridSpec(
        num_scalar_prefetch=1,
        grid=(B, S // S_BLK),
        in_specs=[
            pl.BlockSpec((1, H, D), lambda b, s, kv: (b, 0, 0)),
            pl.BlockSpec((1, S_BLK, D), lambda b, s, kv: (b, s, 0)),
            pl.BlockSpec((1, 1, H), lambda b, s, kv: (b, 0, 0)),
            pl.BlockSpec((D, D), lambda b, s, kv: (0, 0)),
        ],
        out_specs=pl.BlockSpec((1, 1, 1, S_BLK), lambda b, s, kv: (b, s, 0, 0)),
    )
    out = pl.pallas_call(
        _logits_body,
        grid_spec=grid_spec,
        out_shape=jax.ShapeDtypeStruct((B, S // S_BLK, 1, S_BLK), jnp.float32),
    )(kv_lens, q, k_cache, weights.reshape(B, 1, H), jnp.asarray(_HAD_CONST))
    return out.reshape(B, S)


def kernel(q, k_cache, weights, kv_lens):
    keys = _compute_keys(q, k_cache, weights, kv_lens)
    topk_vals, topk_idx = jax.lax.top_k(keys, TOPK)
    topk_vals = jnp.where(topk_vals < 0.0, -1e30, topk_vals)
    return topk_vals, topk_idx
